# Initial kernel scaffold; baseline (speedup 1.0000x reference)
import functools

import jax
import jax.numpy as jnp
from jax import lax
from jax.experimental import pallas as pl
from jax.experimental.pallas import tpu as pltpu
from jax.experimental.pallas import tpu_sc as plsc

B, C, N, K = 8, 128, 1024, 16
BLK = 256


def _topk_body(cloud_ref, idx_ref):
    i = pl.program_id(1)
    xf = cloud_ref[0]
    rows = xf[:, pl.ds(i * BLK, BLK)]
    inner = lax.dot_general(
        rows, xf, (((0,), (0,)), ((), ())),
        preferred_element_type=jnp.float32)
    sq = jnp.sum(xf * xf, axis=0)
    sq_rows = sq[pl.ds(i * BLK, BLK)]
    d = sq_rows[:, None] + sq[None, :] - 2.0 * inner

    lane = lax.broadcasted_iota(jnp.int32, (BLK, N), 1)
    cols = []
    for _ in range(K):
        m = jnp.min(d, axis=1)
        cand = jnp.where(d == m[:, None], lane, N)
        amin = jnp.min(cand, axis=1)
        cols.append(amin)
        d = jnp.where(lane == amin[:, None], jnp.inf, d)
    idx_ref[0] = jnp.stack(cols, axis=1)


def _nn_idx(cloud):
    return pl.pallas_call(
        _topk_body,
        grid=(B, N // BLK),
        in_specs=[pl.BlockSpec((1, C, N), lambda b, i: (b, 0, 0))],
        out_specs=pl.BlockSpec((1, BLK, K), lambda b, i: (b, i, 0)),
        out_shape=jax.ShapeDtypeStruct((B, N, K), jnp.int32),
    )(cloud)


_SC_MESH = plsc.VectorSubcoreMesh(core_axis_name="c", subcore_axis_name="s")
_NW = 32
_CPW = C * B // _NW


@functools.partial(
    pl.kernel,
    out_type=jax.ShapeDtypeStruct((B, 2 * C, N * K), jnp.float32),
    mesh=_SC_MESH,
    scratch_types=[
        pltpu.VMEM((N * K,), jnp.int32),
        pltpu.VMEM((N,), jnp.float32),
        pltpu.VMEM((N * K,), jnp.float32),
        pltpu.VMEM((N * K,), jnp.float32),
    ],
)
def _edge_sc(cloud_hbm, idx_hbm, out_hbm, idx_v, row_v, cen_v, edge_v):
    wid = lax.axis_index("s") * 2 + lax.axis_index("c")
    b = wid // (_NW // B)
    c0 = (wid % (_NW // B)) * _CPW
    pltpu.sync_copy(idx_hbm.at[b], idx_v)

    def per_channel(cc, _):
        c = c0 + cc
        pltpu.sync_copy(cloud_hbm.at[b, c], row_v)

        def per_vec(i, _):
            iv = idx_v[pl.ds(i * K, K)]
            nb = plsc.load_gather(row_v, [iv])
            cv = plsc.load_gather(row_v, [jnp.full((K,), 0, jnp.int32) + i])
            cen_v[pl.ds(i * K, K)] = cv
            edge_v[pl.ds(i * K, K)] = nb - cv
            return 0

        lax.fori_loop(0, N, per_vec, 0)
        pltpu.sync_copy(cen_v, out_hbm.at[b, c])
        pltpu.sync_copy(edge_v, out_hbm.at[b, C + c])
        return 0

    lax.fori_loop(0, _CPW, per_channel, 0)


def kernel(cloud):
    idx = _nn_idx(cloud)
    out = _edge_sc(cloud, idx.reshape(B, N * K))
    return out.reshape(B, 2 * C, N, K)


# speedup vs baseline: 292.3317x; 292.3317x over previous
import functools

import jax
import jax.numpy as jnp
from jax import lax
from jax.experimental import pallas as pl
from jax.experimental.pallas import tpu as pltpu
from jax.experimental.pallas import tpu_sc as plsc

B, C, N, K = 8, 128, 1024, 16
BLK = 256


def _topk_body(cloud_ref, idx_ref):
    i = pl.program_id(1)
    xf = cloud_ref[0]
    rows = cloud_ref[0, :, pl.ds(i * BLK, BLK)]
    inner = lax.dot_general(
        rows, xf, (((0,), (0,)), ((), ())),
        preferred_element_type=jnp.float32)
    sq = jnp.sum(xf * xf, axis=0)
    sq_rows = jnp.sum(rows * rows, axis=0)
    d = sq_rows[:, None] + sq[None, :] - 2.0 * inner

    lane = lax.broadcasted_iota(jnp.int32, (BLK, N), 1)
    cols = []
    for _ in range(K):
        m = jnp.min(d, axis=1)
        cand = jnp.where(d == m[:, None], lane, N)
        amin = jnp.min(cand, axis=1)
        cols.append(amin)
        d = jnp.where(lane == amin[:, None], jnp.inf, d)
    idx_ref[0] = jnp.stack(cols, axis=1)


def _nn_idx(cloud):
    return pl.pallas_call(
        _topk_body,
        grid=(B, N // BLK),
        in_specs=[pl.BlockSpec((1, C, N), lambda b, i: (b, 0, 0))],
        out_specs=pl.BlockSpec((1, BLK, K), lambda b, i: (b, i, 0)),
        out_shape=jax.ShapeDtypeStruct((B, N, K), jnp.int32),
    )(cloud)


_NW = 32
_CPW = C * B // _NW


@functools.lru_cache(maxsize=None)
def _edge_sc():
    mesh = plsc.VectorSubcoreMesh(
        core_axis_name="c", subcore_axis_name="s", num_cores=2,
        num_subcores=16)

    @functools.partial(
        pl.kernel,
        out_type=jax.ShapeDtypeStruct((B, 2 * C, N * K), jnp.float32),
        mesh=mesh,
        compiler_params=pltpu.CompilerParams(needs_layout_passes=False),
        scratch_types=[
            pltpu.VMEM((N * K,), jnp.int32),
            pltpu.VMEM((N,), jnp.float32),
            pltpu.VMEM((N * K,), jnp.float32),
            pltpu.VMEM((N * K,), jnp.float32),
        ],
    )
    def edge_sc(cloud_hbm, idx_hbm, out_hbm, idx_v, row_v, cen_v, edge_v):
        wid = lax.axis_index("s") * 2 + lax.axis_index("c")
        b = wid // (_NW // B)
        c0 = (wid % (_NW // B)) * _CPW
        pltpu.sync_copy(idx_hbm.at[b], idx_v)

        def per_channel(cc, _):
            c = c0 + cc
            pltpu.sync_copy(cloud_hbm.at[b, c], row_v)

            def per_vec(i, _):
                iv = idx_v[pl.ds(i * K, K)]
                nb = plsc.load_gather(row_v, [iv])
                cv = plsc.load_gather(row_v, [jnp.full((K,), 0, jnp.int32) + i])
                cen_v[pl.ds(i * K, K)] = cv
                edge_v[pl.ds(i * K, K)] = nb - cv
                return 0

            lax.fori_loop(0, N, per_vec, 0)
            pltpu.sync_copy(cen_v, out_hbm.at[b, c])
            pltpu.sync_copy(edge_v, out_hbm.at[b, C + c])
            return 0

        lax.fori_loop(0, _CPW, per_channel, 0)

    return edge_sc


def kernel(cloud):
    idx = _nn_idx(cloud)
    out = _edge_sc()(cloud, idx.reshape(B, N * K))
    return out.reshape(B, 2 * C, N, K)
